# baseline (speedup 1.0000x reference)
import functools

import jax
import jax.numpy as jnp
import numpy as np
from jax import lax
from jax.experimental import pallas as pl
from jax.experimental.pallas import tpu as pltpu
from jax.experimental.pallas import tpu_sc as plsc

_N, _D, _E, _H, _C = 10000, 128, 320000, 16, 2
_NC, _NS = 2, 16
_NW = _NC * _NS
_EPW = _E // _NW
_EPT = _E // _NW
_R = _N // 8

_mesh = plsc.VectorSubcoreMesh(core_axis_name="c", subcore_axis_name="s")

_KSWAP = np.zeros((128, 128), np.float32)
_PSEL = np.zeros((128, 16), np.float32)
for _a in range(8):
    _KSWAP[_a * 16 + 1, _a * 16 + 0] = 1.0
    _KSWAP[_a * 16 + 0, _a * 16 + 1] = 1.0
    for _j in range(_C):
        _PSEL[_a * 16 + _j, _a * _C + _j] = 1.0


@functools.partial(
    pl.kernel,
    out_type=jax.ShapeDtypeStruct((_NC, _N, 16), jnp.float32),
    mesh=_mesh,
    scratch_types=[
        pltpu.VMEM((_EPT,), jnp.int32),
        pltpu.VMEM((_EPT,), jnp.float32),
        pltpu.VMEM((1000,), jnp.float32),
        pltpu.VMEM((1000, 16), jnp.float32),
        pltpu.VMEM_SHARED((_N,), jnp.float32),
    ],
    compiler_params=pltpu.CompilerParams(use_tc_tiling_on_sc=False),
)
def _deg_kernel(edges, ones_e, zeros_d, deg_out, dst_v, val_v, comp_v, wide_v, deg_sh):
    c = lax.axis_index("c")
    s = lax.axis_index("s")
    base = _E + (c * _NS + s) * _EPT
    pltpu.sync_copy(edges.at[pl.ds(base, _EPT)], dst_v)
    pltpu.sync_copy(ones_e, val_v)

    @pl.when(s < 10)
    def _():
        pltpu.sync_copy(zeros_d.at[pl.ds(s * 1000, 1000)], comp_v)
        pltpu.sync_copy(comp_v, deg_sh.at[pl.ds(s * 1000, 1000)])

    plsc.subcore_barrier()
    pltpu.sync_copy(val_v, deg_sh.at[dst_v], add=True)
    plsc.subcore_barrier()

    @pl.when(s < 10)
    def _():
        pltpu.sync_copy(deg_sh.at[pl.ds(s * 1000, 1000)], comp_v)

        def ebody(g, _):
            v = comp_v[pl.ds(g * 16, 16)]
            for k in range(16):
                wide_v[g * 16 + k, :] = jnp.broadcast_to(
                    lax.slice(v, (k,), (k + 1,)), (16,))
            return 0

        lax.fori_loop(0, 1000 // 16, ebody, 0)
        vtail = comp_v[pl.ds(984, 16)]
        for k in range(8, 16):
            wide_v[984 + k, :] = jnp.broadcast_to(
                lax.slice(vtail, (k,), (k + 1,)), (16,))
        pltpu.sync_copy(wide_v, deg_out.at[c, pl.ds(s * 1000, 1000)])


def _make_agg_kernel(feat_dim, batch, nbatch):
    assert batch * nbatch == _EPW

    @functools.partial(
        pl.kernel,
        out_type=jax.ShapeDtypeStruct((_NC, _N, feat_dim), jnp.float32),
        mesh=_mesh,
        scratch_types=(
            [pltpu.VMEM((batch,), jnp.int32) for _ in range(nbatch)]
            + [pltpu.VMEM((batch,), jnp.int32) for _ in range(nbatch)]
            + [
                pltpu.VMEM((batch, feat_dim), jnp.float32),
                pltpu.VMEM((batch, feat_dim), jnp.float32),
                pltpu.VMEM_SHARED((_N, feat_dim), jnp.float32),
                pltpu.SemaphoreType.DMA,
                pltpu.SemaphoreType.DMA,
            ]
        ),
        compiler_params=pltpu.CompilerParams(use_tc_tiling_on_sc=False),
    )
    def _agg(edges, y, zeros_a, out, *rest):
        src_v = rest[:nbatch]
        dst_v = rest[nbatch:2 * nbatch]
        rows_a, rows_b, acc_sh, sem_a, sem_b = rest[2 * nbatch:]
        bufs = (rows_a, rows_b)
        sems = (sem_a, sem_b)
        c = lax.axis_index("c")
        s = lax.axis_index("s")
        base = (c * _NS + s) * _EPW
        for b in range(nbatch):
            pltpu.sync_copy(edges.at[pl.ds(base + b * batch, batch)], src_v[b])
            pltpu.sync_copy(edges.at[pl.ds(_E + base + b * batch, batch)], dst_v[b])
        zchunk = 1000
        nz = _N // zchunk

        @pl.when(s < nz)
        def _():
            pltpu.sync_copy(zeros_a.at[pl.ds(s * zchunk, zchunk)],
                            rows_a.at[pl.ds(0, zchunk)])
            pltpu.sync_copy(rows_a.at[pl.ds(0, zchunk)],
                            acc_sh.at[pl.ds(s * zchunk, zchunk)])
        plsc.subcore_barrier()
        cps = [pltpu.async_copy(y.at[src_v[0]], bufs[0], sems[0])]
        for b in range(nbatch):
            cps[b].wait()
            if b + 1 < nbatch:
                nxt = (b + 1) % 2
                cps.append(pltpu.async_copy(y.at[src_v[b + 1]], bufs[nxt], sems[nxt]))
            pltpu.sync_copy(bufs[b % 2], acc_sh.at[dst_v[b]], add=True)
        plsc.subcore_barrier()

        @pl.when(s < 10)
        def _():
            pltpu.sync_copy(acc_sh.at[pl.ds(s * 1000, 1000)],
                            rows_a.at[pl.ds(0, 1000)])
            pltpu.sync_copy(rows_a.at[pl.ds(0, 1000)],
                            out.at[c, pl.ds(s * 1000, 1000)])

    return _agg


_agg_kernel = _make_agg_kernel(_H, 2000, 5)


def _mm1_body(x8_ref, w8_ref, y_ref):
    y_ref[...] = jnp.dot(x8_ref[...], w8_ref[...],
                         preferred_element_type=jnp.float32)


def _scale_body(yraw_ref, deg_ref, y_ref, d_ref):
    d = deg_ref[0] + deg_ref[1] + 1.0
    d_ref[...] = d
    y_ref[...] = yraw_ref[...] * lax.rsqrt(d)


def _mm2_body(acc_ref, y_ref, deg_ref, b1_ref, w2k_ref, z_ref):
    dis = lax.rsqrt(deg_ref[...])
    a = acc_ref[0] + acc_ref[1] + y_ref[...]
    h = jnp.maximum(a * dis + b1_ref[...], 0.0)
    z_ref[...] = jnp.dot(h, w2k_ref[...], preferred_element_type=jnp.float32) * dis


def _out_body(acc_ref, z_ref, deg_ref, b2_ref, k_ref, p_ref, o_ref):
    dis = lax.rsqrt(deg_ref[...])
    o = (acc_ref[0] + acc_ref[1] + z_ref[...]) * dis + b2_ref[...]
    osw = jnp.dot(o, k_ref[...], preferred_element_type=jnp.float32)
    m = jnp.maximum(o, osw)
    sm = jnp.exp(o - m) + jnp.exp(osw - m)
    r = (o - m) - jnp.log(sm)
    o_ref[...] = jnp.dot(r, p_ref[...], preferred_element_type=jnp.float32)


def _fs(shape):
    return pl.BlockSpec(shape, lambda: tuple(0 for _ in shape))


def kernel(x, edge_index, W1, b1, W2, b2):
    f32 = jnp.float32
    edges = edge_index.astype(jnp.int32).reshape(2 * _E)
    ones_e = jnp.ones((_EPT,), f32)
    zeros_d = jnp.zeros((_N,), f32)
    zeros1 = jnp.zeros((_N, _H), f32)
    eye8 = jnp.eye(8, dtype=f32)

    deg16 = _deg_kernel(edges, ones_e, zeros_d)
    degp = deg16.reshape(_NC, _R, 128)

    x8 = x.reshape(_R, 8 * _D)
    W8 = jnp.kron(eye8, W1)
    yraw = pl.pallas_call(
        _mm1_body,
        in_specs=[_fs((_R, 8 * _D)), _fs((8 * _D, 128))],
        out_specs=_fs((_R, 128)),
        out_shape=jax.ShapeDtypeStruct((_R, 128), f32),
    )(x8, W8)
    y128, deg128 = pl.pallas_call(
        _scale_body,
        in_specs=[_fs((_R, 128)), _fs((_NC, _R, 128))],
        out_specs=(_fs((_R, 128)), _fs((_R, 128))),
        out_shape=(jax.ShapeDtypeStruct((_R, 128), f32),
                   jax.ShapeDtypeStruct((_R, 128), f32)),
    )(yraw, degp)

    acc1 = _agg_kernel(edges, y128.reshape(_N, _H), zeros1)
    acc1p = acc1.reshape(_NC, _R, 128)

    W2k = jnp.kron(eye8, jnp.pad(W2, ((0, 0), (0, _H - _C))))
    b1t = jnp.tile(b1, 8).reshape(1, 128)
    z128 = pl.pallas_call(
        _mm2_body,
        in_specs=[_fs((_NC, _R, 128)), _fs((_R, 128)), _fs((_R, 128)),
                  _fs((1, 128)), _fs((128, 128))],
        out_specs=_fs((_R, 128)),
        out_shape=jax.ShapeDtypeStruct((_R, 128), f32),
    )(acc1p, y128, deg128, b1t, W2k)

    acc2 = _agg_kernel(edges, z128.reshape(_N, _H), zeros1)
    acc2p = acc2.reshape(_NC, _R, 128)

    b2t = jnp.tile(jnp.pad(b2, (0, _H - _C)), 8).reshape(1, 128)
    out16 = pl.pallas_call(
        _out_body,
        in_specs=[_fs((_NC, _R, 128)), _fs((_R, 128)), _fs((_R, 128)),
                  _fs((1, 128)), _fs((128, 128)), _fs((128, 16))],
        out_specs=_fs((_R, 16)),
        out_shape=jax.ShapeDtypeStruct((_R, 16), f32),
    )(acc2p, z128, deg128, b2t, jnp.asarray(_KSWAP), jnp.asarray(_PSEL))

    return out16.reshape(_N, _C)


# speedup vs baseline: 1.0091x; 1.0091x over previous
import functools

import jax
import jax.numpy as jnp
import numpy as np
from jax import lax
from jax.experimental import pallas as pl
from jax.experimental.pallas import tpu as pltpu
from jax.experimental.pallas import tpu_sc as plsc

_N, _D, _E, _H, _C = 10000, 128, 320000, 16, 2
_NC, _NS = 2, 16
_NW = _NC * _NS
_EPW = _E // _NW
_EPT = _E // _NS
_HALF = _N // _NC
_DPAD = 5120
_R = _N // 8

_mesh = plsc.VectorSubcoreMesh(core_axis_name="c", subcore_axis_name="s")

_KSWAP = np.zeros((128, 128), np.float32)
_PSEL = np.zeros((128, 16), np.float32)
for _a in range(8):
    _KSWAP[_a * 16 + 1, _a * 16 + 0] = 1.0
    _KSWAP[_a * 16 + 0, _a * 16 + 1] = 1.0
    for _j in range(_C):
        _PSEL[_a * 16 + _j, _a * _C + _j] = 1.0


@functools.partial(
    pl.kernel,
    out_type=jax.ShapeDtypeStruct((_N, 16), jnp.float32),
    mesh=_mesh,
    scratch_types=[
        pltpu.VMEM((_EPT,), jnp.int32),
        pltpu.VMEM((_EPT,), jnp.int32),
        pltpu.VMEM((_EPT,), jnp.float32),
        pltpu.VMEM((1000,), jnp.float32),
        pltpu.VMEM((1000, 16), jnp.float32),
        pltpu.VMEM_SHARED((_DPAD,), jnp.float32),
    ],
    compiler_params=pltpu.CompilerParams(use_tc_tiling_on_sc=False),
)
def _deg_kernel(edges, zeros_d, deg_out, dst_v, idx_v, val_v, comp_v, wide_v, deg_sh):
    c = lax.axis_index("c")
    s = lax.axis_index("s")
    lo = c * _HALF
    pltpu.sync_copy(edges.at[pl.ds(_E + s * _EPT, _EPT)], dst_v)

    @pl.when(s == 0)
    def _():
        pltpu.sync_copy(zeros_d, val_v.at[pl.ds(0, _DPAD)])
        pltpu.sync_copy(val_v.at[pl.ds(0, _DPAD)], deg_sh)

    def tbody(i, _):
        v = dst_v[pl.ds(i * 16, 16)]
        inr = (v >= lo) & (v < lo + _HALF)
        idx_v[pl.ds(i * 16, 16)] = jnp.where(inr, v - lo, v & 4095)
        val_v[pl.ds(i * 16, 16)] = jnp.where(inr, jnp.float32(1.0), jnp.float32(0.0))
        return 0

    lax.fori_loop(0, _EPT // 16, tbody, 0)
    plsc.subcore_barrier()
    pltpu.sync_copy(val_v, deg_sh.at[idx_v], add=True)
    plsc.subcore_barrier()

    @pl.when(s < 5)
    def _():
        pltpu.sync_copy(deg_sh.at[pl.ds(s * 1000, 1000)], comp_v)

        def ebody(g, _):
            v = comp_v[pl.ds(g * 16, 16)]
            for k in range(16):
                wide_v[g * 16 + k, :] = jnp.broadcast_to(
                    lax.slice(v, (k,), (k + 1,)), (16,))
            return 0

        lax.fori_loop(0, 1000 // 16, ebody, 0)
        vtail = comp_v[pl.ds(984, 16)]
        for k in range(8, 16):
            wide_v[984 + k, :] = jnp.broadcast_to(
                lax.slice(vtail, (k,), (k + 1,)), (16,))
        pltpu.sync_copy(wide_v, deg_out.at[pl.ds(lo + s * 1000, 1000)])


def _make_agg_kernel(feat_dim, batch, nbatch):
    assert batch * nbatch == _EPW

    @functools.partial(
        pl.kernel,
        out_type=jax.ShapeDtypeStruct((_NC, _N, feat_dim), jnp.float32),
        mesh=_mesh,
        scratch_types=(
            [pltpu.VMEM((batch,), jnp.int32) for _ in range(nbatch)]
            + [pltpu.VMEM((batch,), jnp.int32) for _ in range(nbatch)]
            + [
                pltpu.VMEM((batch, feat_dim), jnp.float32),
                pltpu.VMEM((batch, feat_dim), jnp.float32),
                pltpu.VMEM_SHARED((_N, feat_dim), jnp.float32),
                pltpu.SemaphoreType.DMA,
                pltpu.SemaphoreType.DMA,
            ]
        ),
        compiler_params=pltpu.CompilerParams(use_tc_tiling_on_sc=False),
    )
    def _agg(edges, y, zeros_a, out, *rest):
        src_v = rest[:nbatch]
        dst_v = rest[nbatch:2 * nbatch]
        rows_a, rows_b, acc_sh, sem_a, sem_b = rest[2 * nbatch:]
        bufs = (rows_a, rows_b)
        sems = (sem_a, sem_b)
        c = lax.axis_index("c")
        s = lax.axis_index("s")
        base = (c * _NS + s) * _EPW
        for b in range(nbatch):
            pltpu.sync_copy(edges.at[pl.ds(base + b * batch, batch)], src_v[b])
            pltpu.sync_copy(edges.at[pl.ds(_E + base + b * batch, batch)], dst_v[b])
        zchunk = 1000
        nz = _N // zchunk

        @pl.when(s < nz)
        def _():
            pltpu.sync_copy(zeros_a.at[pl.ds(s * zchunk, zchunk)],
                            rows_a.at[pl.ds(0, zchunk)])
            pltpu.sync_copy(rows_a.at[pl.ds(0, zchunk)],
                            acc_sh.at[pl.ds(s * zchunk, zchunk)])
        plsc.subcore_barrier()
        cps = [pltpu.async_copy(y.at[src_v[0]], bufs[0], sems[0])]
        for b in range(nbatch):
            cps[b].wait()
            if b + 1 < nbatch:
                nxt = (b + 1) % 2
                cps.append(pltpu.async_copy(y.at[src_v[b + 1]], bufs[nxt], sems[nxt]))
            pltpu.sync_copy(bufs[b % 2], acc_sh.at[dst_v[b]], add=True)
        plsc.subcore_barrier()

        @pl.when(s < 10)
        def _():
            pltpu.sync_copy(acc_sh.at[pl.ds(s * 1000, 1000)],
                            rows_a.at[pl.ds(0, 1000)])
            pltpu.sync_copy(rows_a.at[pl.ds(0, 1000)],
                            out.at[c, pl.ds(s * 1000, 1000)])

    return _agg


_agg_kernel = _make_agg_kernel(_H, 2000, 5)


def _mm1_body(x8_ref, w8_ref, y_ref):
    y_ref[...] = jnp.dot(x8_ref[...], w8_ref[...],
                         preferred_element_type=jnp.float32)


def _scale_body(yraw_ref, deg_ref, y_ref):
    y_ref[...] = yraw_ref[...] * lax.rsqrt(deg_ref[...] + 1.0)


def _mm2_body(acc_ref, y_ref, deg_ref, b1_ref, w2k_ref, z_ref):
    dis = lax.rsqrt(deg_ref[...] + 1.0)
    a = acc_ref[0] + acc_ref[1] + y_ref[...]
    h = jnp.maximum(a * dis + b1_ref[...], 0.0)
    z_ref[...] = jnp.dot(h, w2k_ref[...], preferred_element_type=jnp.float32) * dis


def _out_body(acc_ref, z_ref, deg_ref, b2_ref, k_ref, p_ref, o_ref):
    dis = lax.rsqrt(deg_ref[...] + 1.0)
    o = (acc_ref[0] + acc_ref[1] + z_ref[...]) * dis + b2_ref[...]
    osw = jnp.dot(o, k_ref[...], preferred_element_type=jnp.float32)
    m = jnp.maximum(o, osw)
    sm = jnp.exp(o - m) + jnp.exp(osw - m)
    r = (o - m) - jnp.log(sm)
    o_ref[...] = jnp.dot(r, p_ref[...], preferred_element_type=jnp.float32)


def _fs(shape):
    return pl.BlockSpec(shape, lambda: tuple(0 for _ in shape))


def kernel(x, edge_index, W1, b1, W2, b2):
    f32 = jnp.float32
    edges = edge_index.astype(jnp.int32).reshape(2 * _E)
    zeros_d = jnp.zeros((_DPAD,), f32)
    zeros1 = jnp.zeros((_N, _H), f32)
    eye8 = jnp.eye(8, dtype=f32)

    deg16 = _deg_kernel(edges, zeros_d)
    deg128 = deg16.reshape(_R, 128)

    x8 = x.reshape(_R, 8 * _D)
    W8 = jnp.kron(eye8, W1)
    yraw = pl.pallas_call(
        _mm1_body,
        in_specs=[_fs((_R, 8 * _D)), _fs((8 * _D, 128))],
        out_specs=_fs((_R, 128)),
        out_shape=jax.ShapeDtypeStruct((_R, 128), f32),
    )(x8, W8)
    y128 = pl.pallas_call(
        _scale_body,
        in_specs=[_fs((_R, 128)), _fs((_R, 128))],
        out_specs=_fs((_R, 128)),
        out_shape=jax.ShapeDtypeStruct((_R, 128), f32),
    )(yraw, deg128)

    acc1 = _agg_kernel(edges, y128.reshape(_N, _H), zeros1)
    acc1p = acc1.reshape(_NC, _R, 128)

    W2k = jnp.kron(eye8, jnp.pad(W2, ((0, 0), (0, _H - _C))))
    b1t = jnp.tile(b1, 8).reshape(1, 128)
    z128 = pl.pallas_call(
        _mm2_body,
        in_specs=[_fs((_NC, _R, 128)), _fs((_R, 128)), _fs((_R, 128)),
                  _fs((1, 128)), _fs((128, 128))],
        out_specs=_fs((_R, 128)),
        out_shape=jax.ShapeDtypeStruct((_R, 128), f32),
    )(acc1p, y128, deg128, b1t, W2k)

    acc2 = _agg_kernel(edges, z128.reshape(_N, _H), zeros1)
    acc2p = acc2.reshape(_NC, _R, 128)

    b2t = jnp.tile(jnp.pad(b2, (0, _H - _C)), 8).reshape(1, 128)
    out16 = pl.pallas_call(
        _out_body,
        in_specs=[_fs((_NC, _R, 128)), _fs((_R, 128)), _fs((_R, 128)),
                  _fs((1, 128)), _fs((128, 128)), _fs((128, 16))],
        out_specs=_fs((_R, 16)),
        out_shape=jax.ShapeDtypeStruct((_R, 16), f32),
    )(acc2p, z128, deg128, b2t, jnp.asarray(_KSWAP), jnp.asarray(_PSEL))

    return out16.reshape(_N, _C)
